# baseline (speedup 1.0000x reference)
import jax
import jax.numpy as jnp
from jax.experimental import pallas as pl
from jax.experimental.pallas import tpu as pltpu


def _pick_bm(n, target=400):
    best = None
    for bm in range(8, min(n, target) + 1, 8):
        if n % bm == 0:
            best = bm
    return best if best is not None else n


def _make_fused_kernel(bm, in_dim, hid, emb_d, proj):
    hb = bm // 2

    def _fused(x_ref, adjt_ref, adjb_ref, wcat_ref, bcat_ref,
               emb_ref, z_ref, y1_s, y2_s):
        p = pl.program_id(0)
        i = pl.program_id(1)
        f32 = jnp.float32
        bf16 = jnp.bfloat16
        adjt_b = adjt_ref[...].astype(bf16)
        adjb_b = adjb_ref[...].astype(bf16)

        @pl.when(jnp.logical_and(p == 0, i == 0))
        def _():
            w1 = wcat_ref[0:in_dim, :]
            b1 = bcat_ref[0:1, :]
            y1_s[...] = (
                jnp.dot(x_ref[...], w1, preferred_element_type=f32) + b1
            ).astype(bf16)

        @pl.when(p == 0)
        def _():
            w2 = wcat_ref[in_dim:in_dim + hid, :]
            b2 = bcat_ref[1:2, :]
            y1 = y1_s[...]
            ht = jnp.maximum(
                jnp.dot(adjt_b, y1, preferred_element_type=f32), 0.0)
            hb_ = jnp.maximum(
                jnp.dot(adjb_b, y1, preferred_element_type=f32), 0.0)
            y2_s[pl.ds(i * bm, hb), :] = (
                jnp.dot(ht, w2, preferred_element_type=f32) + b2
            ).astype(bf16)
            y2_s[pl.ds(i * bm + hb, hb), :] = (
                jnp.dot(hb_, w2, preferred_element_type=f32) + b2
            ).astype(bf16)

        @pl.when(p == 1)
        def _():
            w3 = wcat_ref[in_dim + hid:in_dim + hid + emb_d, :]
            b3 = bcat_ref[2:3, :]
            w4 = wcat_ref[in_dim + hid + emb_d:in_dim + hid + emb_d + proj, :]
            b4 = bcat_ref[3:4, :]
            y2 = y2_s[...]
            embt = jnp.dot(adjt_b, y2, preferred_element_type=f32)
            embb = jnp.dot(adjb_b, y2, preferred_element_type=f32)
            emb_ref[0:hb, :] = embt
            emb_ref[hb:bm, :] = embb
            tt = jnp.maximum(
                jnp.dot(embt, w3, preferred_element_type=f32) + b3, 0.0)
            tb = jnp.maximum(
                jnp.dot(embb, w3, preferred_element_type=f32) + b3, 0.0)
            z_ref[0:hb, :] = jnp.dot(tt, w4, preferred_element_type=f32) + b4
            z_ref[hb:bm, :] = jnp.dot(tb, w4, preferred_element_type=f32) + b4

    return _fused


@jax.jit
def kernel(x, Adj_, W1, b1, W2, b2, W3, b3, W4, b4):
    n, in_dim = x.shape
    hid = W1.shape[1]
    emb_d = W2.shape[1]
    proj = W4.shape[1]
    f32 = jnp.float32

    wcat = jnp.concatenate([W1, W2, W3, W4], axis=0)
    bcat = jnp.stack([b1, b2, b3, b4], axis=0)
    wrows = wcat.shape[0]

    bm = _pick_bm(n)
    grid = (2, n // bm)
    hb = bm // 2

    const2 = lambda r, c: pl.BlockSpec((r, c), lambda p, i: (0, 0))
    adjt_spec = pl.BlockSpec((hb, n), lambda p, i: (2 * i, 0))
    adjb_spec = pl.BlockSpec((hb, n), lambda p, i: (2 * i + 1, 0))
    out_spec = lambda d: pl.BlockSpec((bm, d), lambda p, i: (i * p, 0))

    emb, z = pl.pallas_call(
        _make_fused_kernel(bm, in_dim, hid, emb_d, proj),
        grid=grid,
        in_specs=[
            const2(n, in_dim),
            adjt_spec,
            adjb_spec,
            const2(wrows, emb_d),
            const2(4, emb_d),
        ],
        out_specs=[out_spec(emb_d), out_spec(proj)],
        out_shape=[
            jax.ShapeDtypeStruct((n, emb_d), f32),
            jax.ShapeDtypeStruct((n, proj), f32),
        ],
        scratch_shapes=[
            pltpu.VMEM((n, hid), jnp.bfloat16),
            pltpu.VMEM((n, emb_d), jnp.bfloat16),
        ],
        compiler_params=pltpu.CompilerParams(
            dimension_semantics=("arbitrary", "arbitrary"),
        ),
    )(x, Adj_, Adj_, wcat, bcat)

    return (z, emb)


# speedup vs baseline: 1.0248x; 1.0248x over previous
import jax
import jax.numpy as jnp
from jax.experimental import pallas as pl
from jax.experimental.pallas import tpu as pltpu


def _pick_bm(n, target=400):
    best = None
    for bm in range(8, min(n, target) + 1, 8):
        if n % bm == 0:
            best = bm
    return best if best is not None else n


def _prep_kernel(x_ref, w_ref, b_ref, o_ref):
    o_ref[...] = (
        jnp.dot(x_ref[...], w_ref[...], preferred_element_type=jnp.float32)
        + b_ref[...]
    ).astype(jnp.bfloat16)


def _make_fused_kernel(bm, nb, k_stash):
    ilast = nb - 1

    def _fused(y1_ref, adj_ref, w2_ref, b2_ref,
               w3_ref, b3_ref, w4_ref, b4_ref,
               emb_ref, z_ref, y2_s, cast_s):
        p = pl.program_id(0)
        i = pl.program_id(1)
        f32 = jnp.float32
        bf16 = jnp.bfloat16
        slot = jnp.minimum(i, k_stash) * bm

        def head(emb):
            emb_ref[...] = emb
            t = jnp.maximum(
                jnp.dot(emb, w3_ref[...], preferred_element_type=f32)
                + b3_ref[...],
                0.0,
            )
            z_ref[...] = (
                jnp.dot(t, w4_ref[...], preferred_element_type=f32)
                + b4_ref[...]
            )

        @pl.when(p == 0)
        def _():
            cast_s[pl.ds(slot, bm), :] = adj_ref[...].astype(bf16)
            tile = cast_s[pl.ds(slot, bm), :]
            h = jnp.dot(tile, y1_ref[...], preferred_element_type=f32)
            h = jnp.maximum(h, 0.0)
            y2_s[pl.ds(i * bm, bm), :] = (
                jnp.dot(h, w2_ref[...], preferred_element_type=f32)
                + b2_ref[...]
            ).astype(bf16)

            @pl.when(i == ilast)
            def _():
                head(jnp.dot(tile, y2_s[...], preferred_element_type=f32))

        if k_stash > 0:
            @pl.when(jnp.logical_and(p == 1,
                                     jnp.logical_and(i < k_stash, i < ilast)))
            def _():
                a = cast_s[0:bm, :]
                head(jnp.dot(a, y2_s[...], preferred_element_type=f32))

        @pl.when(jnp.logical_and(p == 1,
                                 jnp.logical_and(i >= k_stash, i < ilast)))
        def _():
            cast_s[pl.ds(k_stash * bm, bm), :] = adj_ref[...].astype(bf16)
            a = cast_s[pl.ds(k_stash * bm, bm), :]
            head(jnp.dot(a, y2_s[...], preferred_element_type=f32))

    return _fused


@jax.jit
def kernel(x, Adj_, W1, b1, W2, b2, W3, b3, W4, b4):
    n, in_dim = x.shape
    hid = W1.shape[1]
    emb_d = W2.shape[1]
    proj = W4.shape[1]
    f32 = jnp.float32

    b1r = b1.reshape(1, -1)
    b2r = b2.reshape(1, -1)
    b3r = b3.reshape(1, -1)
    b4r = b4.reshape(1, -1)

    y1 = pl.pallas_call(
        _prep_kernel,
        out_shape=jax.ShapeDtypeStruct((n, hid), jnp.bfloat16),
    )(x, W1, b1r)

    bm = _pick_bm(n)
    nb = n // bm
    ilast = nb - 1
    park = max(ilast - 1, 0)
    k_stash = max(0, min(1, nb - 1))
    grid = (2, nb)

    def adj_idx(p, i):
        p1 = jnp.where(i < k_stash, ilast, jnp.where(i == ilast, park, i))
        return (jnp.where(p == 0, i, p1), 0)

    def out_idx(p, i):
        return (jnp.where(p == 0, ilast, jnp.minimum(i, park)), 0)

    const2 = lambda r, c: pl.BlockSpec((r, c), lambda p, i: (0, 0))

    emb, z = pl.pallas_call(
        _make_fused_kernel(bm, nb, k_stash),
        grid=grid,
        in_specs=[
            const2(n, hid),
            pl.BlockSpec((bm, n), adj_idx),
            const2(hid, emb_d),
            const2(1, emb_d),
            const2(emb_d, proj),
            const2(1, proj),
            const2(proj, proj),
            const2(1, proj),
        ],
        out_specs=[
            pl.BlockSpec((bm, emb_d), out_idx),
            pl.BlockSpec((bm, proj), out_idx),
        ],
        out_shape=[
            jax.ShapeDtypeStruct((n, emb_d), f32),
            jax.ShapeDtypeStruct((n, proj), f32),
        ],
        scratch_shapes=[
            pltpu.VMEM((n, emb_d), jnp.bfloat16),
            pltpu.VMEM(((k_stash + 1) * bm, n), jnp.bfloat16),
        ],
        compiler_params=pltpu.CompilerParams(
            dimension_semantics=("arbitrary", "arbitrary"),
            vmem_limit_bytes=64 * 1024 * 1024,
        ),
    )(y1, Adj_, W2, b2r, W3, b3r, W4, b4r)

    return (z, emb)


# speedup vs baseline: 1.0395x; 1.0144x over previous
import jax
import jax.numpy as jnp
from jax.experimental import pallas as pl
from jax.experimental.pallas import tpu as pltpu


def _pick_bm(n, target=400):
    best = None
    for bm in range(8, min(n, target) + 1, 8):
        if n % bm == 0:
            best = bm
    return best if best is not None else n


def _prep_kernel(x_ref, w_ref, b_ref, o_ref):
    o_ref[...] = (
        jnp.dot(x_ref[...], w_ref[...], preferred_element_type=jnp.float32)
        + b_ref[...]
    ).astype(jnp.bfloat16)


def _make_fused_kernel(bm, nb, k_stash):
    ilast = nb - 1

    def _fused(y1_ref, adj_ref, w2_ref, b2_ref,
               w3_ref, b3_ref, w4_ref, b4_ref,
               emb_ref, z_ref, y2_s, cast_s):
        p = pl.program_id(0)
        i = pl.program_id(1)
        f32 = jnp.float32
        bf16 = jnp.bfloat16
        slot = jnp.minimum(i, k_stash) * bm

        def head(emb):
            emb_ref[...] = emb
            t = jnp.maximum(
                jnp.dot(emb, w3_ref[...], preferred_element_type=f32)
                + b3_ref[...],
                0.0,
            )
            z_ref[...] = (
                jnp.dot(t, w4_ref[...], preferred_element_type=f32)
                + b4_ref[...]
            )

        def layer1_tail(h):
            h = jnp.maximum(h, 0.0)
            y2_s[pl.ds(i * bm, bm), :] = (
                jnp.dot(h, w2_ref[...], preferred_element_type=f32)
                + b2_ref[...]
            ).astype(bf16)

        @pl.when(p == 0)
        def _():
            if k_stash > 0:
                @pl.when(i < k_stash)
                def _():
                    cast_s[pl.ds(slot, bm), :] = adj_ref[...].astype(bf16)
                    tile = cast_s[pl.ds(slot, bm), :]
                    layer1_tail(jnp.dot(tile, y1_ref[...],
                                        preferred_element_type=f32))

            @pl.when(i >= k_stash)
            def _():
                y1f = y1_ref[...].astype(f32)
                layer1_tail(jnp.dot(adj_ref[...], y1f,
                                    preferred_element_type=f32))

                @pl.when(i == ilast)
                def _():
                    y2f = y2_s[...].astype(f32)
                    head(jnp.dot(adj_ref[...], y2f,
                                 preferred_element_type=f32))

        if k_stash > 0:
            @pl.when(jnp.logical_and(p == 1,
                                     jnp.logical_and(i < k_stash, i < ilast)))
            def _():
                a = cast_s[pl.ds(slot, bm), :]
                head(jnp.dot(a, y2_s[...], preferred_element_type=f32))

        @pl.when(jnp.logical_and(p == 1,
                                 jnp.logical_and(i >= k_stash, i < ilast)))
        def _():
            y2f = y2_s[...].astype(f32)
            head(jnp.dot(adj_ref[...], y2f, preferred_element_type=f32))

    return _fused


@jax.jit
def kernel(x, Adj_, W1, b1, W2, b2, W3, b3, W4, b4):
    n, in_dim = x.shape
    hid = W1.shape[1]
    emb_d = W2.shape[1]
    proj = W4.shape[1]
    f32 = jnp.float32

    b1r = b1.reshape(1, -1)
    b2r = b2.reshape(1, -1)
    b3r = b3.reshape(1, -1)
    b4r = b4.reshape(1, -1)

    y1 = pl.pallas_call(
        _prep_kernel,
        out_shape=jax.ShapeDtypeStruct((n, hid), jnp.bfloat16),
    )(x, W1, b1r)

    bm = _pick_bm(n)
    nb = n // bm
    ilast = nb - 1
    park = max(ilast - 1, 0)
    k_stash = max(0, min(2, nb - 1))
    grid = (2, nb)

    def adj_idx(p, i):
        p1 = jnp.where(i < k_stash, ilast, jnp.where(i == ilast, park, i))
        return (jnp.where(p == 0, i, p1), 0)

    def out_idx(p, i):
        return (jnp.where(p == 0, ilast, jnp.minimum(i, park)), 0)

    const2 = lambda r, c: pl.BlockSpec((r, c), lambda p, i: (0, 0))

    emb, z = pl.pallas_call(
        _make_fused_kernel(bm, nb, k_stash),
        grid=grid,
        in_specs=[
            const2(n, hid),
            pl.BlockSpec((bm, n), adj_idx),
            const2(hid, emb_d),
            const2(1, emb_d),
            const2(emb_d, proj),
            const2(1, proj),
            const2(proj, proj),
            const2(1, proj),
        ],
        out_specs=[
            pl.BlockSpec((bm, emb_d), out_idx),
            pl.BlockSpec((bm, proj), out_idx),
        ],
        out_shape=[
            jax.ShapeDtypeStruct((n, emb_d), f32),
            jax.ShapeDtypeStruct((n, proj), f32),
        ],
        scratch_shapes=[
            pltpu.VMEM((n, emb_d), jnp.bfloat16),
            pltpu.VMEM((max(k_stash, 1) * bm, n), jnp.bfloat16),
        ],
        compiler_params=pltpu.CompilerParams(
            dimension_semantics=("arbitrary", "arbitrary"),
            vmem_limit_bytes=64 * 1024 * 1024,
        ),
    )(y1, Adj_, W2, b2r, W3, b3r, W4, b4r)

    return (z, emb)
